# Initial kernel scaffold; baseline (speedup 1.0000x reference)
import jax
import jax.numpy as jnp
from jax.experimental import pallas as pl


def kernel(x, adj, W1, b1, W2, b2):
    raise NotImplementedError("write your pallas kernel here")


# speedup vs baseline: 133.2161x; 133.2161x over previous
import jax
import jax.numpy as jnp
from jax import lax
from jax.experimental import pallas as pl
from jax.experimental.pallas import tpu as pltpu

_B, _N = 48, 200
_NN = _N * _N
_K = int(_NN * 0.2)
_H1, _H2 = 128, 128


def _sortable(y):
    return jnp.where(y >= 0, y, y ^ jnp.int32(0x7FFFFFFF))


def _thresh_body(x_ref, t_ref):
    y = _sortable(lax.bitcast_convert_type(x_ref[...], jnp.int32))

    sign = jnp.int32(-2147483647 - 1)

    def step(i, u):
        bit = lax.shift_left(jnp.int32(1), jnp.int32(31) - i)
        up = u | bit
        tp = up ^ sign
        cnt = jnp.sum((y >= tp).astype(jnp.float32), axis=1, keepdims=True)
        return jnp.where(cnt >= jnp.float32(_K), up, u)

    u0 = jnp.zeros((_B, 1), jnp.int32)
    t_ref[...] = lax.fori_loop(0, 32, step, u0) ^ sign


def _gcn_body(thr_ref, x_ref, w1_ref, b1_ref, w2_ref, b2_ref, o_ref):
    xb = x_ref[0]
    t = thr_ref[pl.program_id(0)]
    xt = xb.T
    yt = _sortable(lax.bitcast_convert_type(xt, jnp.int32))
    st = jnp.where(yt >= t, xt, 0.0)
    deg = jnp.sum(st, axis=1, keepdims=True) + 1.0
    dis = jnp.where(deg > 0, lax.rsqrt(jnp.maximum(deg, 1e-12)), 0.0)
    d2 = dis * dis
    h0 = jnp.dot(xb, w1_ref[...], preferred_element_type=jnp.float32)
    a1 = (dis * jnp.dot(st, dis * h0, preferred_element_type=jnp.float32)
          + d2 * h0 + b1_ref[...])
    h1 = jnp.maximum(a1, 0.0)
    g1 = jnp.dot(h1, w2_ref[...], preferred_element_type=jnp.float32)
    a2 = (dis * jnp.dot(st, dis * g1, preferred_element_type=jnp.float32)
          + d2 * g1 + b2_ref[...])
    o_ref[0, 0, :] = jnp.mean(a2, axis=0)


def kernel(x, adj, W1, b1, W2, b2):
    del adj
    thr = pl.pallas_call(
        _thresh_body,
        out_shape=jax.ShapeDtypeStruct((_B, 1), jnp.int32),
    )(x.reshape(_B, _NN))
    out = pl.pallas_call(
        _gcn_body,
        grid=(_B,),
        in_specs=[
            pl.BlockSpec((_B,), lambda b: (0,), memory_space=pltpu.SMEM),
            pl.BlockSpec((1, _N, _N), lambda b: (b, 0, 0)),
            pl.BlockSpec((_N, _H1), lambda b: (0, 0)),
            pl.BlockSpec((1, _H1), lambda b: (0, 0)),
            pl.BlockSpec((_H1, _H2), lambda b: (0, 0)),
            pl.BlockSpec((1, _H2), lambda b: (0, 0)),
        ],
        out_specs=pl.BlockSpec((1, 1, _H2), lambda b: (b, 0, 0)),
        out_shape=jax.ShapeDtypeStruct((_B, 1, _H2), jnp.float32),
    )(thr.reshape(_B), x, W1, b1.reshape(1, _H1), W2, b2.reshape(1, _H2))
    return out.reshape(_B, _H2)
